# baseline (speedup 1.0000x reference)
import functools
import jax
import jax.numpy as jnp
from jax import lax
from jax.experimental import pallas as pl
from jax.experimental.pallas import tpu as pltpu
from jax.experimental.pallas import tpu_sc as plsc

EMB = 64
NC = 2
NS = 16
CH = 128
ROW = 128
NBUF = 4


@functools.partial(jax.jit, static_argnames=("batch",))
def _sc_gather(idx16, du2, di2, lu2, li2, batch):
    nw = NC * NS
    b_per_w = batch // nw
    n_chunks = b_per_w // CH
    n_str = 4 * n_chunks
    mesh = plsc.VectorSubcoreMesh(core_axis_name="c", subcore_axis_name="s")

    @functools.partial(
        pl.kernel,
        mesh=mesh,
        compiler_params=pltpu.CompilerParams(use_tc_tiling_on_sc=False),
        out_type=[jax.ShapeDtypeStruct((batch, ROW), jnp.float32)
                  for _ in range(4)],
        scratch_types=[
            pltpu.VMEM((n_str, CH), jnp.int32),
        ] + [pltpu.VMEM((CH, ROW), jnp.float32) for _ in range(NBUF)] + [
            pltpu.SemaphoreType.DMA,
            pltpu.SemaphoreType.DMA,
        ],
    )
    def gather_kernel(idx_hbm, du_hbm, di_hbm, lu_hbm, li_hbm,
                      gu_hbm, gi_hbm, wu_hbm, wi_hbm,
                      idx_v, *rest):
        bufs = list(rest[:NBUF])
        sem, osem = rest[NBUF], rest[NBUF + 1]
        tabs = [du_hbm, di_hbm, lu_hbm, li_hbm]
        outs = [gu_hbm, gi_hbm, wu_hbm, wi_hbm]
        wid = lax.axis_index("s") * NC + lax.axis_index("c")
        base = wid * b_per_w
        pltpu.sync_copy(idx_hbm.at[wid], idx_v)

        def gth(k):
            t, j = divmod(k, n_chunks)
            return (tabs[t].at[idx_v.at[k]], bufs[k % NBUF], sem)

        def put(k):
            t, j = divmod(k, n_chunks)
            dst = outs[t].at[pl.ds(base + j * CH, CH), :]
            return (bufs[k % NBUF], dst, osem)

        for k in range(n_str):
            if k >= NBUF:
                pltpu.make_async_copy(*put(k - NBUF)).wait()
            pltpu.async_copy(*gth(k))
            if k >= NBUF - 1:
                kk = k - NBUF + 1
                pltpu.make_async_copy(*gth(kk)).wait()
                pltpu.async_copy(*put(kk))
        for k in range(n_str - NBUF + 1, n_str):
            pltpu.make_async_copy(*gth(k)).wait()
            pltpu.async_copy(*put(k))
        for k in range(n_str - NBUF, n_str):
            pltpu.make_async_copy(*put(k)).wait()

    return gather_kernel(idx16, du2, di2, lu2, li2)


def _mlp_body(gu, gi, wu, wi, offu, offi, wcu, wci,
              w1u, w1i, b1, w2, b2, out):
    blk = gu.shape[0]
    ue = jnp.where(offu[...] == 1, gu[:, EMB:], gu[:, :EMB])
    ie = jnp.where(offi[...] == 1, gi[:, EMB:], gi[:, :EMB])
    h = (jnp.dot(ue, w1u[...], preferred_element_type=jnp.float32)
         + jnp.dot(ie, w1i[...], preferred_element_type=jnp.float32))
    h = jnp.maximum(h + b1[...], 0.0)
    deep = jnp.dot(h, w2[...], preferred_element_type=jnp.float32)
    lanes = lax.broadcasted_iota(jnp.int32, (blk, ROW), 1)
    wide_u = jnp.sum(jnp.where(lanes == wcu[...], wu[...], 0.0),
                     axis=1, keepdims=True)
    wide_i = jnp.sum(jnp.where(lanes == wci[...], wi[...], 0.0),
                     axis=1, keepdims=True)
    out[...] = deep + wide_u + wide_i + b2[...]


@jax.jit
def _tc_mlp(gu, gi, wu, wi, offu, offi, wcu, wci, w1u, w1i, b1, w2, b2):
    batch = gu.shape[0]
    blk = 2048
    grid = (batch // blk,)
    row_spec = pl.BlockSpec((blk, ROW), lambda i: (i, 0))
    col_spec = pl.BlockSpec((blk, 1), lambda i: (i, 0))
    return pl.pallas_call(
        _mlp_body,
        grid=grid,
        in_specs=[
            row_spec, row_spec, row_spec, row_spec,
            col_spec, col_spec, col_spec, col_spec,
            pl.BlockSpec((EMB, EMB), lambda i: (0, 0)),
            pl.BlockSpec((EMB, EMB), lambda i: (0, 0)),
            pl.BlockSpec((1, EMB), lambda i: (0, 0)),
            pl.BlockSpec((EMB, 1), lambda i: (0, 0)),
            pl.BlockSpec((1, 1), lambda i: (0, 0)),
        ],
        out_specs=pl.BlockSpec((blk, 1), lambda i: (i, 0)),
        out_shape=jax.ShapeDtypeStruct((batch, 1), jnp.float32),
    )(gu, gi, wu, wi, offu, offi, wcu, wci, w1u, w1i, b1, w2, b2)


def kernel(user_idx, item_idx, linear_user, linear_item, deep_user,
           deep_item, W1, b1, W2, b2):
    batch = user_idx.shape[0]
    n = deep_user.shape[0]
    nw = NC * NS
    b_per_w = batch // nw
    n_chunks = b_per_w // CH
    nrow_lin = (n + ROW - 1) // ROW
    ui = user_idx.astype(jnp.int32)
    ii = item_idx.astype(jnp.int32)
    idx16 = jnp.concatenate([
        (ui >> 1).reshape(nw, n_chunks, CH),
        (ii >> 1).reshape(nw, n_chunks, CH),
        (ui >> 7).reshape(nw, n_chunks, CH),
        (ii >> 7).reshape(nw, n_chunks, CH),
    ], axis=1)
    du2 = deep_user.reshape(n // 2, ROW)
    di2 = deep_item.reshape(n // 2, ROW)
    lu2 = jnp.pad(linear_user.reshape(-1),
                  (0, nrow_lin * ROW - n)).reshape(nrow_lin, ROW)
    li2 = jnp.pad(linear_item.reshape(-1),
                  (0, nrow_lin * ROW - n)).reshape(nrow_lin, ROW)
    gu, gi, wu, wi = _sc_gather(idx16, du2, di2, lu2, li2, batch)
    out = _tc_mlp(gu, gi, wu, wi,
                  (ui & 1).reshape(batch, 1), (ii & 1).reshape(batch, 1),
                  (ui & 127).reshape(batch, 1), (ii & 127).reshape(batch, 1),
                  W1[:, :EMB].T, W1[:, EMB:].T, b1.reshape(1, EMB),
                  W2.reshape(EMB, 1), b2.reshape(1, 1))
    return out


# speedup vs baseline: 1.0095x; 1.0095x over previous
import functools
import jax
import jax.numpy as jnp
from jax import lax
from jax.experimental import pallas as pl
from jax.experimental.pallas import tpu as pltpu
from jax.experimental.pallas import tpu_sc as plsc

EMB = 64
NC = 2
NS = 16
CH = 128
ROW = 128
NBUF = 4


@functools.partial(jax.jit, static_argnames=("batch",))
def _sc_gather(idx16, du, di, lu2, li2, batch):
    nw = NC * NS
    b_per_w = batch // nw
    n_chunks = b_per_w // CH
    n_str = 4 * n_chunks
    mesh = plsc.VectorSubcoreMesh(core_axis_name="c", subcore_axis_name="s")

    @functools.partial(
        pl.kernel,
        mesh=mesh,
        compiler_params=pltpu.CompilerParams(use_tc_tiling_on_sc=False),
        out_type=[
            jax.ShapeDtypeStruct((batch, EMB), jnp.float32),
            jax.ShapeDtypeStruct((batch, EMB), jnp.float32),
            jax.ShapeDtypeStruct((batch, ROW), jnp.float32),
            jax.ShapeDtypeStruct((batch, ROW), jnp.float32),
        ],
        scratch_types=[
            pltpu.VMEM((n_str, CH), jnp.int32),
        ] + [pltpu.VMEM((CH, EMB), jnp.float32) for _ in range(NBUF)]
          + [pltpu.VMEM((CH, ROW), jnp.float32) for _ in range(NBUF)] + [
            pltpu.SemaphoreType.DMA,
            pltpu.SemaphoreType.DMA,
        ],
    )
    def gather_kernel(idx_hbm, du_hbm, di_hbm, lu_hbm, li_hbm,
                      gu_hbm, gi_hbm, wu_hbm, wi_hbm,
                      idx_v, *rest):
        dbufs = list(rest[:NBUF])
        wbufs = list(rest[NBUF:2 * NBUF])
        sem, osem = rest[2 * NBUF], rest[2 * NBUF + 1]
        tabs = [du_hbm, di_hbm, lu_hbm, li_hbm]
        outs = [gu_hbm, gi_hbm, wu_hbm, wi_hbm]
        wid = lax.axis_index("s") * NC + lax.axis_index("c")
        base = wid * b_per_w
        pltpu.sync_copy(idx_hbm.at[wid], idx_v)

        def buf(k):
            t = k // n_chunks
            return (dbufs if t < 2 else wbufs)[k % NBUF]

        def gth(k):
            t = k // n_chunks
            return (tabs[t].at[idx_v.at[k]], buf(k), sem)

        def put(k):
            t, j = divmod(k, n_chunks)
            dst = outs[t].at[pl.ds(base + j * CH, CH), :]
            return (buf(k), dst, osem)

        for k in range(n_str):
            if k >= NBUF:
                pltpu.make_async_copy(*put(k - NBUF)).wait()
            pltpu.async_copy(*gth(k))
            if k >= NBUF - 1:
                kk = k - NBUF + 1
                pltpu.make_async_copy(*gth(kk)).wait()
                pltpu.async_copy(*put(kk))
        for k in range(n_str - NBUF + 1, n_str):
            pltpu.make_async_copy(*gth(k)).wait()
            pltpu.async_copy(*put(k))
        for k in range(n_str - NBUF, n_str):
            pltpu.make_async_copy(*put(k)).wait()

    return gather_kernel(idx16, du, di, lu2, li2)


def _mlp_body(gu, gi, wu, wi, wcu, wci, w1u, w1i, b1, w2, b2, out):
    blk = gu.shape[0]
    h = (jnp.dot(gu[...], w1u[...], preferred_element_type=jnp.float32)
         + jnp.dot(gi[...], w1i[...], preferred_element_type=jnp.float32))
    h = jnp.maximum(h + b1[...], 0.0)
    deep = jnp.dot(h, w2[...], preferred_element_type=jnp.float32)
    lanes = lax.broadcasted_iota(jnp.int32, (blk, ROW), 1)
    wide_u = jnp.sum(jnp.where(lanes == wcu[...], wu[...], 0.0),
                     axis=1, keepdims=True)
    wide_i = jnp.sum(jnp.where(lanes == wci[...], wi[...], 0.0),
                     axis=1, keepdims=True)
    out[...] = deep + wide_u + wide_i + b2[...]


@jax.jit
def _tc_mlp(gu, gi, wu, wi, wcu, wci, w1u, w1i, b1, w2, b2):
    batch = gu.shape[0]
    blk = 2048
    grid = (batch // blk,)
    emb_spec = pl.BlockSpec((blk, EMB), lambda i: (i, 0))
    row_spec = pl.BlockSpec((blk, ROW), lambda i: (i, 0))
    col_spec = pl.BlockSpec((blk, 1), lambda i: (i, 0))
    return pl.pallas_call(
        _mlp_body,
        grid=grid,
        in_specs=[
            emb_spec, emb_spec, row_spec, row_spec,
            col_spec, col_spec,
            pl.BlockSpec((EMB, EMB), lambda i: (0, 0)),
            pl.BlockSpec((EMB, EMB), lambda i: (0, 0)),
            pl.BlockSpec((1, EMB), lambda i: (0, 0)),
            pl.BlockSpec((EMB, 1), lambda i: (0, 0)),
            pl.BlockSpec((1, 1), lambda i: (0, 0)),
        ],
        out_specs=pl.BlockSpec((blk, 1), lambda i: (i, 0)),
        out_shape=jax.ShapeDtypeStruct((batch, 1), jnp.float32),
    )(gu, gi, wu, wi, wcu, wci, w1u, w1i, b1, w2, b2)


def kernel(user_idx, item_idx, linear_user, linear_item, deep_user,
           deep_item, W1, b1, W2, b2):
    batch = user_idx.shape[0]
    n = deep_user.shape[0]
    nw = NC * NS
    n_chunks = batch // nw // CH
    nrow_lin = (n + ROW - 1) // ROW
    ui = user_idx.astype(jnp.int32)
    ii = item_idx.astype(jnp.int32)
    idx16 = jnp.concatenate([
        ui.reshape(nw, n_chunks, CH),
        ii.reshape(nw, n_chunks, CH),
        (ui >> 7).reshape(nw, n_chunks, CH),
        (ii >> 7).reshape(nw, n_chunks, CH),
    ], axis=1)
    lu2 = jnp.pad(linear_user.reshape(-1),
                  (0, nrow_lin * ROW - n)).reshape(nrow_lin, ROW)
    li2 = jnp.pad(linear_item.reshape(-1),
                  (0, nrow_lin * ROW - n)).reshape(nrow_lin, ROW)
    gu, gi, wu, wi = _sc_gather(idx16, deep_user, deep_item, lu2, li2, batch)
    out = _tc_mlp(gu, gi, wu, wi,
                  (ui & 127).reshape(batch, 1), (ii & 127).reshape(batch, 1),
                  W1[:, :EMB].T, W1[:, EMB:].T, b1.reshape(1, EMB),
                  W2.reshape(EMB, 1), b2.reshape(1, 1))
    return out
